# baseline (speedup 1.0000x reference)
import functools

import jax
import jax.numpy as jnp
import numpy as np
from jax import lax
from jax.experimental import pallas as pl
from jax.experimental.pallas import tpu as pltpu
from jax.experimental.pallas import tpu_sc as plsc

_bf16 = jnp.bfloat16

_PERM = np.empty((128,), np.int32)
for _g in range(4):
    for _t in range(16):
        _PERM[32 * _g + 2 * _t] = 32 * _g + _t
        _PERM[32 * _g + 2 * _t + 1] = 32 * _g + 16 + _t

N = 10000
E = 320000
DX = 128
DE = 16
G = 64
L = 3
NC = 40
EPS = 1e-5

NCORE = 2
NSUB = 16
NWORK = NCORE * NSUB
CH = 40
SUP = 32
KCH = 8
CPW = SUP * KCH
EPW = CH * CPW
E_PAD = NWORK * EPW
NHB = 4
NPB = 4
LA = 2
NAGG = 10112
RPT = NAGG // NSUB

BN = 1000
BE = 2048

_f32 = jnp.float32


def _h0_body(x_ref, w_ref, scale_ref, bias_ref, o_ref):
    h = jnp.dot(x_ref[...], w_ref[...], preferred_element_type=_f32)
    o_ref[...] = jnp.maximum(h * scale_ref[...] + bias_ref[...], 0.0)


def _h0(x, w, scale, bias):
    return pl.pallas_call(
        _h0_body,
        grid=(N // BN,),
        in_specs=[
            pl.BlockSpec((BN, DX), lambda i: (i, 0)),
            pl.BlockSpec((DX, DX), lambda i: (0, 0)),
            pl.BlockSpec((1, DX), lambda i: (0, 0)),
            pl.BlockSpec((1, DX), lambda i: (0, 0)),
        ],
        out_specs=pl.BlockSpec((BN, DX), lambda i: (i, 0)),
        out_shape=jax.ShapeDtypeStruct((N, DX), _f32),
    )(x, w, scale, bias)


def _edge_body(ea_ref, wp0, wp1, wp2, we0, be0, we1, be1, p0_ref, p1_ref, p2_ref):
    ea0 = ea_ref[...]
    p0_ref[...] = jnp.dot(ea0, wp0[...], preferred_element_type=_f32)
    ea1 = jnp.dot(ea0, we0[...], preferred_element_type=_f32) + be0[...]
    p1_ref[...] = jnp.dot(ea1, wp1[...], preferred_element_type=_f32)
    ea2 = jnp.dot(ea1, we1[...], preferred_element_type=_f32) + be1[...]
    p2_ref[...] = jnp.dot(ea2, wp2[...], preferred_element_type=_f32)


def _edge_chain(ea_pad, Wp, We, be):
    ew = pl.BlockSpec((DE, DX), lambda i: (0, 0))
    sw = pl.BlockSpec((DE, DE), lambda i: (0, 0))
    bw = pl.BlockSpec((1, DE), lambda i: (0, 0))
    pspec = pl.BlockSpec((BE, DX), lambda i: (i, 0))
    pshape = jax.ShapeDtypeStruct((E_PAD, DX), _f32)
    return pl.pallas_call(
        _edge_body,
        grid=(E_PAD // BE,),
        in_specs=[pl.BlockSpec((BE, DE), lambda i: (i, 0)), ew, ew, ew,
                  sw, bw, sw, bw],
        out_specs=[pspec, pspec, pspec],
        out_shape=[pshape, pshape, pshape],
    )(ea_pad, Wp[0], Wp[1], Wp[2], We[0], be[0].reshape(1, DE),
      We[1], be[1].reshape(1, DE))


def _sc_layer_body(h_hbm, p_hbm, src_hbm, dst_hbm, zeros_hbm, out_hbm,
                   src_v, dst_v, hbuf, pbuf, agg_sh,
                   gsem, psem, ssem, isem):
    cc = lax.axis_index("c")
    ss = lax.axis_index("s")
    wid = ss * NCORE + cc
    pbase = wid * EPW
    pltpu.sync_copy(zeros_hbm.at[pl.ds(ss * RPT, RPT)],
                    agg_sh.at[pl.ds(ss * RPT, RPT)])
    plsc.subcore_barrier()

    def issue_idx(s, sl):
        pltpu.async_copy(src_hbm.at[wid, pl.ds(s * KCH, KCH)],
                         src_v.at[sl], isem)
        pltpu.async_copy(dst_hbm.at[wid, pl.ds(s * KCH, KCH)],
                         dst_v.at[sl], isem)

    def wait_idx():
        pltpu.make_async_copy(src_hbm.at[wid, pl.ds(0, KCH)],
                              src_v.at[0], isem).wait()
        pltpu.make_async_copy(dst_hbm.at[wid, pl.ds(0, KCH)],
                              dst_v.at[0], isem).wait()

    def issue_gp(c, sl, k, hslot, pslot):
        pltpu.async_copy(h_hbm.at[src_v.at[sl, k]], hbuf.at[hslot], gsem)
        pltpu.async_copy(p_hbm.at[pl.ds(pbase + c * CH, CH)],
                         pbuf.at[pslot], psem)

    def wait_gp(hslot, pslot):
        pltpu.make_async_copy(h_hbm.at[src_v.at[0, 0]], hbuf.at[hslot],
                              gsem).wait()
        pltpu.make_async_copy(p_hbm.at[pl.ds(0, CH)], pbuf.at[pslot],
                              psem).wait()

    def wait_scatter(pslot):
        pltpu.make_async_copy(pbuf.at[pslot], agg_sh.at[dst_v.at[0, 0]],
                              ssem).wait()

    def compute(hslot, pslot):
        @plsc.parallel_loop(0, CH * (DX // 16), unroll=8)
        def _(j):
            r = j >> 3
            off = (j & 7) * 16
            hv = hbuf[hslot, r, pl.ds(off, 16)]
            pv = pbuf[pslot, r, pl.ds(off, 16)]
            pbuf[pslot, r, pl.ds(off, 16)] = jnp.maximum(hv + pv, 0.0)

    def do_chunk(c, sl, k, first_super, last_super):
        hslot, pslot = k % NHB, k % NPB
        if not (first_super and k < LA):
            wait_scatter((k + LA) % NPB)
        if k == KCH - LA and not last_super:
            wait_idx()
        if not last_super or k < KCH - LA:
            nsl = sl if k < KCH - LA else 1 - sl
            issue_gp(c + LA, nsl, (k + LA) % KCH,
                     (k + LA) % NHB, (k + LA) % NPB)
        wait_gp(hslot, pslot)
        compute(hslot, pslot)
        pltpu.async_copy(pbuf.at[pslot], agg_sh.at[dst_v.at[sl, k]], ssem,
                         add=True)

    def run_super(s, sl, first_super, last_super):
        for k in range(KCH):
            do_chunk(s * KCH + k if not first_super else k,
                     sl, k, first_super, last_super)
            if k == 1 and not first_super and not last_super:
                issue_idx(s + 1, 1 - sl)

    pltpu.sync_copy(src_hbm.at[wid, pl.ds(0, KCH)], src_v.at[0])
    pltpu.sync_copy(dst_hbm.at[wid, pl.ds(0, KCH)], dst_v.at[0])
    issue_idx(1, 1)
    for c in range(LA):
        issue_gp(c, 0, c, c % NHB, c % NPB)
    run_super(0, 0, True, False)

    def super_body(s, carry):
        sl = lax.rem(s, 2)
        run_super(s, sl, False, False)
        return carry

    lax.fori_loop(1, SUP - 1, super_body, 0)
    run_super(SUP - 1, (SUP - 1) % 2, False, True)
    for k in range(LA):
        wait_scatter((KCH - LA + k) % NPB)
    plsc.subcore_barrier()
    pltpu.sync_copy(agg_sh.at[pl.ds(ss * RPT, RPT)],
                    out_hbm.at[cc, pl.ds(ss * RPT, RPT)])


def _sc_layer(h, p, src3d, dst3d, zeros):
    mesh = plsc.VectorSubcoreMesh(core_axis_name="c", subcore_axis_name="s",
                                  num_cores=NCORE, num_subcores=NSUB)
    f = pl.kernel(
        _sc_layer_body,
        out_type=jax.ShapeDtypeStruct((2, NAGG, DX), _f32),
        mesh=mesh,
        scratch_types=[
            pltpu.VMEM((2, KCH, CH), jnp.int32),
            pltpu.VMEM((2, KCH, CH), jnp.int32),
            pltpu.VMEM((NHB, CH, DX), _f32),
            pltpu.VMEM((NPB, CH, DX), _f32),
            pltpu.VMEM_SHARED((NAGG, DX), _f32),
            pltpu.SemaphoreType.DMA,
            pltpu.SemaphoreType.DMA,
            pltpu.SemaphoreType.DMA,
            pltpu.SemaphoreType.DMA,
        ],
    )
    return f(h, p, src3d, dst3d, zeros)


def _upd_body(h_ref, a0_ref, a1_ref, wc, bc, b3d, hn_ref, em_ref):
    i = pl.program_id(0)
    hn = jnp.dot(h_ref[...] + a0_ref[0] + a1_ref[0], wc[...],
                 preferred_element_type=_f32) + bc[...]
    hn_ref[...] = hn
    onehot = (b3d[0] == lax.broadcasted_iota(jnp.int32, (G, BN), 0)).astype(_f32)
    part = jnp.dot(onehot, hn, preferred_element_type=_f32)

    @pl.when(i == 0)
    def _():
        em_ref[...] = part

    @pl.when(i > 0)
    def _():
        em_ref[...] = em_ref[...] + part


def _update_pool(h, agg2, wc, bc, batch3d):
    return pl.pallas_call(
        _upd_body,
        grid=(N // BN,),
        in_specs=[
            pl.BlockSpec((BN, DX), lambda i: (i, 0)),
            pl.BlockSpec((1, BN, DX), lambda i: (0, i, 0)),
            pl.BlockSpec((1, BN, DX), lambda i: (1, i, 0)),
            pl.BlockSpec((DX, DX), lambda i: (0, 0)),
            pl.BlockSpec((1, DX), lambda i: (0, 0)),
            pl.BlockSpec((1, 1, BN), lambda i: (i, 0, 0)),
        ],
        out_specs=[pl.BlockSpec((BN, DX), lambda i: (i, 0)),
                   pl.BlockSpec((G, DX), lambda i: (0, 0))],
        out_shape=[jax.ShapeDtypeStruct((N, DX), _f32),
                   jax.ShapeDtypeStruct((G, DX), _f32)],
    )(h, agg2, agg2, wc, bc, batch3d)


def _fin_body(e0, e1, e2, s0, t0, s1, t1, s2, t2, w0, w1, w2, bl, o_ref):
    acc = bl[...]
    for eref, s, t, w in ((e0, s0, t0, w0), (e1, s1, t1, w1), (e2, s2, t2, w2)):
        v = jnp.maximum(eref[...] * s[...] + t[...], 0.0)
        acc = acc + jnp.dot(v, w[...], preferred_element_type=_f32)
    o_ref[...] = acc


def _final(embds, scales, betas, wls, bl_pad):
    return pl.pallas_call(
        _fin_body,
        out_shape=jax.ShapeDtypeStruct((G, DX), _f32),
    )(embds[0], embds[1], embds[2],
      scales[0], betas[0], scales[1], betas[1], scales[2], betas[2],
      wls[0], wls[1], wls[2], bl_pad)


def kernel(x, edge_index, edge_attr, batch, W_start, b_start, g_start,
           beta_start, Wp, Wc, bc, We, be, g_embd, beta_embd, W_lin, b_lin):
    inv = 1.0 / jnp.sqrt(jnp.float32(1.0 + EPS))
    scale0 = (g_start * inv).reshape(1, DX)
    bias0 = (b_start * g_start * inv + beta_start).reshape(1, DX)
    h = _h0(x, W_start, scale0, bias0)

    order = jnp.argsort(edge_index[0])
    src_s = jnp.take(edge_index[0], order)
    dst_s = jnp.take(edge_index[1], order)
    ea_s = jnp.take(edge_attr, order, axis=0)
    pad = E_PAD - E
    src_p = jnp.concatenate([src_s, jnp.full((pad,), N - 1, jnp.int32)])
    dst_p = jnp.concatenate([dst_s, jnp.full((pad,), N, jnp.int32)])
    src3d = src_p.reshape(NWORK, CPW, CH)
    dst3d = dst_p.reshape(NWORK, CPW, CH)
    ea_pad = jnp.concatenate([ea_s, jnp.zeros((pad, DE), _f32)])
    zeros = jnp.zeros((NAGG, DX), _f32)
    batch3d = batch.reshape(N // BN, 1, BN)

    p0, p1, p2 = _edge_chain(ea_pad, Wp, We, be)
    ps = (p0, p1, p2)

    emb_scales = [(g_embd[i] * inv).reshape(1, DX) for i in range(L)]
    emb_betas = [beta_embd[i].reshape(1, DX) for i in range(L)]

    embds = []
    for i in range(L):
        agg2 = _sc_layer(h, ps[i], src3d, dst3d, zeros)
        h, em = _update_pool(h, agg2, Wc[i], bc[i].reshape(1, DX), batch3d)
        embds.append(em)

    wls = [jnp.pad(W_lin[i * DX:(i + 1) * DX], ((0, 0), (0, DX - NC)))
           for i in range(L)]
    bl_pad = jnp.pad(b_lin, (0, DX - NC)).reshape(1, DX)
    out_pad = _final(embds, emb_scales, emb_betas, wls, bl_pad)
    return out_pad[:, :NC]


# speedup vs baseline: 1.4038x; 1.4038x over previous
import functools

import jax
import jax.numpy as jnp
import numpy as np
from jax import lax
from jax.experimental import pallas as pl
from jax.experimental.pallas import tpu as pltpu
from jax.experimental.pallas import tpu_sc as plsc

_bf16 = jnp.bfloat16

_PERM = np.empty((128,), np.int32)
for _g in range(4):
    for _t in range(16):
        _PERM[32 * _g + 2 * _t] = 32 * _g + _t
        _PERM[32 * _g + 2 * _t + 1] = 32 * _g + 16 + _t

N = 10000
E = 320000
DX = 128
DE = 16
G = 64
L = 3
NC = 40
EPS = 1e-5

NCORE = 2
NSUB = 16
NWORK = NCORE * NSUB
CH = 40
SUP = 32
KCH = 8
CPW = SUP * KCH
EPW = CH * CPW
E_PAD = NWORK * EPW
NHB = 4
NPB = 4
LA = 2
LAG = 3
NAGG = 10112
RPT = NAGG // NSUB

BN = 1000
BE = 2048

_f32 = jnp.float32


def _h0_body(x_ref, w_ref, scale_ref, bias_ref, o_ref):
    h = jnp.dot(x_ref[...], w_ref[...], preferred_element_type=_f32)
    o_ref[...] = jnp.maximum(h * scale_ref[...] + bias_ref[...], 0.0)


def _h0(x, w, scale, bias):
    return pl.pallas_call(
        _h0_body,
        grid=(N // BN,),
        in_specs=[
            pl.BlockSpec((BN, DX), lambda i: (i, 0)),
            pl.BlockSpec((DX, DX), lambda i: (0, 0)),
            pl.BlockSpec((1, DX), lambda i: (0, 0)),
            pl.BlockSpec((1, DX), lambda i: (0, 0)),
        ],
        out_specs=pl.BlockSpec((BN, DX), lambda i: (i, 0)),
        out_shape=jax.ShapeDtypeStruct((N, DX), _f32),
    )(x, w, scale, bias)


def _edge_body(ea_ref, wp0, wp1, wp2, we0, be0, we1, be1, p0_ref, p1_ref, p2_ref):
    ea0 = ea_ref[...]
    p0_ref[...] = jnp.dot(ea0, wp0[...], preferred_element_type=_f32)
    ea1 = jnp.dot(ea0, we0[...], preferred_element_type=_f32) + be0[...]
    p1_ref[...] = jnp.dot(ea1, wp1[...], preferred_element_type=_f32)
    ea2 = jnp.dot(ea1, we1[...], preferred_element_type=_f32) + be1[...]
    p2_ref[...] = jnp.dot(ea2, wp2[...], preferred_element_type=_f32)


def _edge_chain(ea_pad, Wp, We, be):
    ew = pl.BlockSpec((DE, DX), lambda i: (0, 0))
    sw = pl.BlockSpec((DE, DE), lambda i: (0, 0))
    bw = pl.BlockSpec((1, DE), lambda i: (0, 0))
    pspec = pl.BlockSpec((BE, DX), lambda i: (i, 0))
    pshape = jax.ShapeDtypeStruct((E_PAD, DX), _f32)
    return pl.pallas_call(
        _edge_body,
        grid=(E_PAD // BE,),
        in_specs=[pl.BlockSpec((BE, DE), lambda i: (i, 0)), ew, ew, ew,
                  sw, bw, sw, bw],
        out_specs=[pspec, pspec, pspec],
        out_shape=[pshape, pshape, pshape],
    )(ea_pad, Wp[0], Wp[1], Wp[2], We[0], be[0].reshape(1, DE),
      We[1], be[1].reshape(1, DE))


def _sc_layer_body(h_hbm, p_hbm, src_hbm, dst_hbm, zeros_hbm, out_hbm,
                   src_v, dst_v, hbuf, pbuf, agg_sh,
                   gsem, psem, ssem, isem):
    cc = lax.axis_index("c")
    ss = lax.axis_index("s")
    wid = ss * NCORE + cc
    pbase = wid * EPW
    pltpu.sync_copy(zeros_hbm.at[pl.ds(ss * RPT, RPT)],
                    agg_sh.at[pl.ds(ss * RPT, RPT)])
    plsc.subcore_barrier()

    def issue_idx(s, sl):
        pltpu.async_copy(src_hbm.at[wid, pl.ds(s * KCH, KCH)],
                         src_v.at[sl], isem)
        pltpu.async_copy(dst_hbm.at[wid, pl.ds(s * KCH, KCH)],
                         dst_v.at[sl], isem)

    def wait_idx():
        pltpu.make_async_copy(src_hbm.at[wid, pl.ds(0, KCH)],
                              src_v.at[0], isem).wait()
        pltpu.make_async_copy(dst_hbm.at[wid, pl.ds(0, KCH)],
                              dst_v.at[0], isem).wait()

    def issue_g(sl, k, hslot):
        pltpu.async_copy(h_hbm.at[src_v.at[sl, k]], hbuf.at[hslot], gsem)

    def issue_p(c, pslot):
        pltpu.async_copy(p_hbm.at[pl.ds(pbase + c * CH, CH)],
                         pbuf.at[pslot], psem)

    def wait_gp(hslot, pslot):
        pltpu.make_async_copy(h_hbm.at[src_v.at[0, 0]], hbuf.at[hslot],
                              gsem).wait()
        pltpu.make_async_copy(p_hbm.at[pl.ds(0, CH)], pbuf.at[pslot],
                              psem).wait()

    def wait_scatter(pslot):
        pltpu.make_async_copy(pbuf.at[pslot], agg_sh.at[dst_v.at[0, 0]],
                              ssem).wait()

    def compute(hslot, pslot):
        @plsc.parallel_loop(0, CH * (DX // 16), unroll=8)
        def _(j):
            r = j >> 3
            off = (j & 7) * 16
            hv = hbuf[hslot, r, pl.ds(off, 16)]
            pv = pbuf[pslot, r, pl.ds(off, 16)]
            pbuf[pslot, r, pl.ds(off, 16)] = jnp.maximum(hv + pv, 0.0)

    def do_chunk(c, sl, k, first_super, last_super):
        hslot, pslot = k % NHB, k % NPB
        if not (first_super and k < LA):
            wait_scatter((k + LA) % NPB)
        if k == KCH - LAG and not last_super:
            wait_idx()
        if not last_super or k < KCH - LAG:
            gsl = sl if k < KCH - LAG else 1 - sl
            issue_g(gsl, (k + LAG) % KCH, (k + LAG) % NHB)
        if not last_super or k < KCH - LA:
            issue_p(c + LA, (k + LA) % NPB)
        wait_gp(hslot, pslot)
        compute(hslot, pslot)
        pltpu.async_copy(pbuf.at[pslot], agg_sh.at[dst_v.at[sl, k]], ssem,
                         add=True)

    def run_super(s, sl, first_super, last_super):
        for k in range(KCH):
            do_chunk(s * KCH + k if not first_super else k,
                     sl, k, first_super, last_super)
            if k == 1 and not first_super and not last_super:
                issue_idx(s + 1, 1 - sl)

    pltpu.sync_copy(src_hbm.at[wid, pl.ds(0, KCH)], src_v.at[0])
    pltpu.sync_copy(dst_hbm.at[wid, pl.ds(0, KCH)], dst_v.at[0])
    issue_idx(1, 1)
    for c in range(LAG):
        issue_g(0, c, c % NHB)
    for c in range(LA):
        issue_p(c, c % NPB)
    run_super(0, 0, True, False)

    def super_body(s, carry):
        sl = lax.rem(s, 2)
        run_super(s, sl, False, False)
        return carry

    lax.fori_loop(1, SUP - 1, super_body, 0)
    run_super(SUP - 1, (SUP - 1) % 2, False, True)
    for k in range(LA):
        wait_scatter((KCH - LA + k) % NPB)
    plsc.subcore_barrier()
    pltpu.sync_copy(agg_sh.at[pl.ds(ss * RPT, RPT)],
                    out_hbm.at[cc, pl.ds(ss * RPT, RPT)])


def _sc_layer(h, p, src3d, dst3d, zeros):
    mesh = plsc.VectorSubcoreMesh(core_axis_name="c", subcore_axis_name="s",
                                  num_cores=NCORE, num_subcores=NSUB)
    f = pl.kernel(
        _sc_layer_body,
        out_type=jax.ShapeDtypeStruct((2, NAGG, DX), _f32),
        mesh=mesh,
        scratch_types=[
            pltpu.VMEM((2, KCH, CH), jnp.int32),
            pltpu.VMEM((2, KCH, CH), jnp.int32),
            pltpu.VMEM((NHB, CH, DX), _f32),
            pltpu.VMEM((NPB, CH, DX), _f32),
            pltpu.VMEM_SHARED((NAGG, DX), _f32),
            pltpu.SemaphoreType.DMA,
            pltpu.SemaphoreType.DMA,
            pltpu.SemaphoreType.DMA,
            pltpu.SemaphoreType.DMA,
        ],
    )
    return f(h, p, src3d, dst3d, zeros)


def _upd_body(h_ref, a0_ref, a1_ref, wc, bc, b3d, hn_ref, em_ref):
    i = pl.program_id(0)
    hn = jnp.dot(h_ref[...] + a0_ref[0] + a1_ref[0], wc[...],
                 preferred_element_type=_f32) + bc[...]
    hn_ref[...] = hn
    onehot = (b3d[0] == lax.broadcasted_iota(jnp.int32, (G, BN), 0)).astype(_f32)
    part = jnp.dot(onehot, hn, preferred_element_type=_f32)

    @pl.when(i == 0)
    def _():
        em_ref[...] = part

    @pl.when(i > 0)
    def _():
        em_ref[...] = em_ref[...] + part


def _update_pool(h, agg2, wc, bc, batch3d):
    return pl.pallas_call(
        _upd_body,
        grid=(N // BN,),
        in_specs=[
            pl.BlockSpec((BN, DX), lambda i: (i, 0)),
            pl.BlockSpec((1, BN, DX), lambda i: (0, i, 0)),
            pl.BlockSpec((1, BN, DX), lambda i: (1, i, 0)),
            pl.BlockSpec((DX, DX), lambda i: (0, 0)),
            pl.BlockSpec((1, DX), lambda i: (0, 0)),
            pl.BlockSpec((1, 1, BN), lambda i: (i, 0, 0)),
        ],
        out_specs=[pl.BlockSpec((BN, DX), lambda i: (i, 0)),
                   pl.BlockSpec((G, DX), lambda i: (0, 0))],
        out_shape=[jax.ShapeDtypeStruct((N, DX), _f32),
                   jax.ShapeDtypeStruct((G, DX), _f32)],
    )(h, agg2, agg2, wc, bc, batch3d)


def _fin_body(e0, e1, e2, s0, t0, s1, t1, s2, t2, w0, w1, w2, bl, o_ref):
    acc = bl[...]
    for eref, s, t, w in ((e0, s0, t0, w0), (e1, s1, t1, w1), (e2, s2, t2, w2)):
        v = jnp.maximum(eref[...] * s[...] + t[...], 0.0)
        acc = acc + jnp.dot(v, w[...], preferred_element_type=_f32)
    o_ref[...] = acc


def _final(embds, scales, betas, wls, bl_pad):
    return pl.pallas_call(
        _fin_body,
        out_shape=jax.ShapeDtypeStruct((G, DX), _f32),
    )(embds[0], embds[1], embds[2],
      scales[0], betas[0], scales[1], betas[1], scales[2], betas[2],
      wls[0], wls[1], wls[2], bl_pad)


def kernel(x, edge_index, edge_attr, batch, W_start, b_start, g_start,
           beta_start, Wp, Wc, bc, We, be, g_embd, beta_embd, W_lin, b_lin):
    inv = 1.0 / jnp.sqrt(jnp.float32(1.0 + EPS))
    scale0 = (g_start * inv).reshape(1, DX)
    bias0 = (b_start * g_start * inv + beta_start).reshape(1, DX)
    h = _h0(x, W_start, scale0, bias0)

    pad = E_PAD - E
    src_p = jnp.concatenate([edge_index[0], jnp.zeros((pad,), jnp.int32)])
    dst_p = jnp.concatenate([edge_index[1], jnp.full((pad,), N, jnp.int32)])
    src3d = src_p.reshape(NWORK, CPW, CH)
    dst3d = dst_p.reshape(NWORK, CPW, CH)
    ea_pad = jnp.concatenate([edge_attr, jnp.zeros((pad, DE), _f32)])
    zeros = jnp.zeros((NAGG, DX), _f32)
    batch3d = batch.reshape(N // BN, 1, BN)

    p0, p1, p2 = _edge_chain(ea_pad, Wp, We, be)
    ps = (p0, p1, p2)

    emb_scales = [(g_embd[i] * inv).reshape(1, DX) for i in range(L)]
    emb_betas = [beta_embd[i].reshape(1, DX) for i in range(L)]

    embds = []
    for i in range(L):
        agg2 = _sc_layer(h, ps[i], src3d, dst3d, zeros)
        h, em = _update_pool(h, agg2, Wc[i], bc[i].reshape(1, DX), batch3d)
        embds.append(em)

    wls = [jnp.pad(W_lin[i * DX:(i + 1) * DX], ((0, 0), (0, DX - NC)))
           for i in range(L)]
    bl_pad = jnp.pad(b_lin, (0, DX - NC)).reshape(1, DX)
    out_pad = _final(embds, emb_scales, emb_betas, wls, bl_pad)
    return out_pad[:, :NC]


# speedup vs baseline: 1.4103x; 1.0046x over previous
import functools

import jax
import jax.numpy as jnp
import numpy as np
from jax import lax
from jax.experimental import pallas as pl
from jax.experimental.pallas import tpu as pltpu
from jax.experimental.pallas import tpu_sc as plsc

_bf16 = jnp.bfloat16

_PERM = np.empty((128,), np.int32)
for _g in range(4):
    for _t in range(16):
        _PERM[32 * _g + 2 * _t] = 32 * _g + _t
        _PERM[32 * _g + 2 * _t + 1] = 32 * _g + 16 + _t

N = 10000
E = 320000
DX = 128
DE = 16
G = 64
L = 3
NC = 40
EPS = 1e-5

NCORE = 2
NSUB = 16
NWORK = NCORE * NSUB
CH = 80
SUP = 16
KCH = 8
CPW = SUP * KCH
EPW = CH * CPW
E_PAD = NWORK * EPW
NHB = 2
NPB = 2
LA = 1
LAG = 1
NAGG = 10112
RPT = NAGG // NSUB

BN = 1000
BE = 2048

_f32 = jnp.float32


def _h0_body(x_ref, w_ref, scale_ref, bias_ref, o_ref):
    h = jnp.dot(x_ref[...], w_ref[...], preferred_element_type=_f32)
    o_ref[...] = jnp.maximum(h * scale_ref[...] + bias_ref[...], 0.0)


def _h0(x, w, scale, bias):
    return pl.pallas_call(
        _h0_body,
        grid=(N // BN,),
        in_specs=[
            pl.BlockSpec((BN, DX), lambda i: (i, 0)),
            pl.BlockSpec((DX, DX), lambda i: (0, 0)),
            pl.BlockSpec((1, DX), lambda i: (0, 0)),
            pl.BlockSpec((1, DX), lambda i: (0, 0)),
        ],
        out_specs=pl.BlockSpec((BN, DX), lambda i: (i, 0)),
        out_shape=jax.ShapeDtypeStruct((N, DX), _f32),
    )(x, w, scale, bias)


def _edge_body(ea_ref, wp0, wp1, wp2, we0, be0, we1, be1, p0_ref, p1_ref, p2_ref):
    ea0 = ea_ref[...]
    p0_ref[...] = jnp.dot(ea0, wp0[...], preferred_element_type=_f32)
    ea1 = jnp.dot(ea0, we0[...], preferred_element_type=_f32) + be0[...]
    p1_ref[...] = jnp.dot(ea1, wp1[...], preferred_element_type=_f32)
    ea2 = jnp.dot(ea1, we1[...], preferred_element_type=_f32) + be1[...]
    p2_ref[...] = jnp.dot(ea2, wp2[...], preferred_element_type=_f32)


def _edge_chain(ea_pad, Wp, We, be):
    ew = pl.BlockSpec((DE, DX), lambda i: (0, 0))
    sw = pl.BlockSpec((DE, DE), lambda i: (0, 0))
    bw = pl.BlockSpec((1, DE), lambda i: (0, 0))
    pspec = pl.BlockSpec((BE, DX), lambda i: (i, 0))
    pshape = jax.ShapeDtypeStruct((E_PAD, DX), _f32)
    return pl.pallas_call(
        _edge_body,
        grid=(E_PAD // BE,),
        in_specs=[pl.BlockSpec((BE, DE), lambda i: (i, 0)), ew, ew, ew,
                  sw, bw, sw, bw],
        out_specs=[pspec, pspec, pspec],
        out_shape=[pshape, pshape, pshape],
    )(ea_pad, Wp[0], Wp[1], Wp[2], We[0], be[0].reshape(1, DE),
      We[1], be[1].reshape(1, DE))


def _sc_layer_body(h_hbm, p_hbm, src_hbm, dst_hbm, zeros_hbm, out_hbm,
                   src_v, dst_v, hbuf, pbuf, agg_sh,
                   gsem, psem, ssem, isem):
    cc = lax.axis_index("c")
    ss = lax.axis_index("s")
    wid = ss * NCORE + cc
    pbase = wid * EPW
    pltpu.sync_copy(zeros_hbm.at[pl.ds(ss * RPT, RPT)],
                    agg_sh.at[pl.ds(ss * RPT, RPT)])
    plsc.subcore_barrier()

    def issue_idx(s, sl):
        pltpu.async_copy(src_hbm.at[wid, pl.ds(s * KCH, KCH)],
                         src_v.at[sl], isem)
        pltpu.async_copy(dst_hbm.at[wid, pl.ds(s * KCH, KCH)],
                         dst_v.at[sl], isem)

    def wait_idx():
        pltpu.make_async_copy(src_hbm.at[wid, pl.ds(0, KCH)],
                              src_v.at[0], isem).wait()
        pltpu.make_async_copy(dst_hbm.at[wid, pl.ds(0, KCH)],
                              dst_v.at[0], isem).wait()

    def issue_g(sl, k, hslot):
        pltpu.async_copy(h_hbm.at[src_v.at[sl, k]], hbuf.at[hslot], gsem)

    def issue_p(c, pslot):
        pltpu.async_copy(p_hbm.at[pl.ds(pbase + c * CH, CH)],
                         pbuf.at[pslot], psem)

    def wait_gp(hslot, pslot):
        pltpu.make_async_copy(h_hbm.at[src_v.at[0, 0]], hbuf.at[hslot],
                              gsem).wait()
        pltpu.make_async_copy(p_hbm.at[pl.ds(0, CH)], pbuf.at[pslot],
                              psem).wait()

    def wait_scatter(pslot):
        pltpu.make_async_copy(pbuf.at[pslot], agg_sh.at[dst_v.at[0, 0]],
                              ssem).wait()

    def compute(hslot, pslot):
        @plsc.parallel_loop(0, CH * (DX // 16), unroll=8)
        def _(j):
            r = j >> 3
            off = (j & 7) * 16
            hv = hbuf[hslot, r, pl.ds(off, 16)]
            pv = pbuf[pslot, r, pl.ds(off, 16)]
            pbuf[pslot, r, pl.ds(off, 16)] = jnp.maximum(hv + pv, 0.0)

    def do_chunk(c, sl, k, first_super, last_super):
        hslot, pslot = k % NHB, k % NPB
        if not (first_super and k < LA):
            wait_scatter((k + LA) % NPB)
        if k == KCH - LAG and not last_super:
            wait_idx()
        if not last_super or k < KCH - LAG:
            gsl = sl if k < KCH - LAG else 1 - sl
            issue_g(gsl, (k + LAG) % KCH, (k + LAG) % NHB)
        if not last_super or k < KCH - LA:
            issue_p(c + LA, (k + LA) % NPB)
        wait_gp(hslot, pslot)
        compute(hslot, pslot)
        pltpu.async_copy(pbuf.at[pslot], agg_sh.at[dst_v.at[sl, k]], ssem,
                         add=True)

    def run_super(s, sl, first_super, last_super):
        for k in range(KCH):
            do_chunk(s * KCH + k if not first_super else k,
                     sl, k, first_super, last_super)
            if k == 1 and not first_super and not last_super:
                issue_idx(s + 1, 1 - sl)

    pltpu.sync_copy(src_hbm.at[wid, pl.ds(0, KCH)], src_v.at[0])
    pltpu.sync_copy(dst_hbm.at[wid, pl.ds(0, KCH)], dst_v.at[0])
    issue_idx(1, 1)
    for c in range(LAG):
        issue_g(0, c, c % NHB)
    for c in range(LA):
        issue_p(c, c % NPB)
    run_super(0, 0, True, False)

    def super_body(s, carry):
        sl = lax.rem(s, 2)
        run_super(s, sl, False, False)
        return carry

    lax.fori_loop(1, SUP - 1, super_body, 0)
    run_super(SUP - 1, (SUP - 1) % 2, False, True)
    for k in range(LA):
        wait_scatter((KCH - LA + k) % NPB)
    plsc.subcore_barrier()
    pltpu.sync_copy(agg_sh.at[pl.ds(ss * RPT, RPT)],
                    out_hbm.at[cc, pl.ds(ss * RPT, RPT)])


def _sc_layer(h, p, src3d, dst3d, zeros):
    mesh = plsc.VectorSubcoreMesh(core_axis_name="c", subcore_axis_name="s",
                                  num_cores=NCORE, num_subcores=NSUB)
    f = pl.kernel(
        _sc_layer_body,
        out_type=jax.ShapeDtypeStruct((2, NAGG, DX), _f32),
        mesh=mesh,
        scratch_types=[
            pltpu.VMEM((2, KCH, CH), jnp.int32),
            pltpu.VMEM((2, KCH, CH), jnp.int32),
            pltpu.VMEM((NHB, CH, DX), _f32),
            pltpu.VMEM((NPB, CH, DX), _f32),
            pltpu.VMEM_SHARED((NAGG, DX), _f32),
            pltpu.SemaphoreType.DMA,
            pltpu.SemaphoreType.DMA,
            pltpu.SemaphoreType.DMA,
            pltpu.SemaphoreType.DMA,
        ],
    )
    return f(h, p, src3d, dst3d, zeros)


def _upd_body(h_ref, a0_ref, a1_ref, wc, bc, b3d, hn_ref, em_ref):
    i = pl.program_id(0)
    hn = jnp.dot(h_ref[...] + a0_ref[0] + a1_ref[0], wc[...],
                 preferred_element_type=_f32) + bc[...]
    hn_ref[...] = hn
    onehot = (b3d[0] == lax.broadcasted_iota(jnp.int32, (G, BN), 0)).astype(_f32)
    part = jnp.dot(onehot, hn, preferred_element_type=_f32)

    @pl.when(i == 0)
    def _():
        em_ref[...] = part

    @pl.when(i > 0)
    def _():
        em_ref[...] = em_ref[...] + part


def _update_pool(h, agg2, wc, bc, batch3d):
    return pl.pallas_call(
        _upd_body,
        grid=(N // BN,),
        in_specs=[
            pl.BlockSpec((BN, DX), lambda i: (i, 0)),
            pl.BlockSpec((1, BN, DX), lambda i: (0, i, 0)),
            pl.BlockSpec((1, BN, DX), lambda i: (1, i, 0)),
            pl.BlockSpec((DX, DX), lambda i: (0, 0)),
            pl.BlockSpec((1, DX), lambda i: (0, 0)),
            pl.BlockSpec((1, 1, BN), lambda i: (i, 0, 0)),
        ],
        out_specs=[pl.BlockSpec((BN, DX), lambda i: (i, 0)),
                   pl.BlockSpec((G, DX), lambda i: (0, 0))],
        out_shape=[jax.ShapeDtypeStruct((N, DX), _f32),
                   jax.ShapeDtypeStruct((G, DX), _f32)],
    )(h, agg2, agg2, wc, bc, batch3d)


def _fin_body(e0, e1, e2, s0, t0, s1, t1, s2, t2, w0, w1, w2, bl, o_ref):
    acc = bl[...]
    for eref, s, t, w in ((e0, s0, t0, w0), (e1, s1, t1, w1), (e2, s2, t2, w2)):
        v = jnp.maximum(eref[...] * s[...] + t[...], 0.0)
        acc = acc + jnp.dot(v, w[...], preferred_element_type=_f32)
    o_ref[...] = acc


def _final(embds, scales, betas, wls, bl_pad):
    return pl.pallas_call(
        _fin_body,
        out_shape=jax.ShapeDtypeStruct((G, DX), _f32),
    )(embds[0], embds[1], embds[2],
      scales[0], betas[0], scales[1], betas[1], scales[2], betas[2],
      wls[0], wls[1], wls[2], bl_pad)


def kernel(x, edge_index, edge_attr, batch, W_start, b_start, g_start,
           beta_start, Wp, Wc, bc, We, be, g_embd, beta_embd, W_lin, b_lin):
    inv = 1.0 / jnp.sqrt(jnp.float32(1.0 + EPS))
    scale0 = (g_start * inv).reshape(1, DX)
    bias0 = (b_start * g_start * inv + beta_start).reshape(1, DX)
    h = _h0(x, W_start, scale0, bias0)

    pad = E_PAD - E
    src_p = jnp.concatenate([edge_index[0], jnp.zeros((pad,), jnp.int32)])
    dst_p = jnp.concatenate([edge_index[1], jnp.full((pad,), N, jnp.int32)])
    src3d = src_p.reshape(NWORK, CPW, CH)
    dst3d = dst_p.reshape(NWORK, CPW, CH)
    ea_pad = jnp.concatenate([edge_attr, jnp.zeros((pad, DE), _f32)])
    zeros = jnp.zeros((NAGG, DX), _f32)
    batch3d = batch.reshape(N // BN, 1, BN)

    p0, p1, p2 = _edge_chain(ea_pad, Wp, We, be)
    ps = (p0, p1, p2)

    emb_scales = [(g_embd[i] * inv).reshape(1, DX) for i in range(L)]
    emb_betas = [beta_embd[i].reshape(1, DX) for i in range(L)]

    embds = []
    for i in range(L):
        agg2 = _sc_layer(h, ps[i], src3d, dst3d, zeros)
        h, em = _update_pool(h, agg2, Wc[i], bc[i].reshape(1, DX), batch3d)
        embds.append(em)

    wls = [jnp.pad(W_lin[i * DX:(i + 1) * DX], ((0, 0), (0, DX - NC)))
           for i in range(L)]
    bl_pad = jnp.pad(b_lin, (0, DX - NC)).reshape(1, DX)
    out_pad = _final(embds, emb_scales, emb_betas, wls, bl_pad)
    return out_pad[:, :NC]


# speedup vs baseline: 1.4107x; 1.0003x over previous
import jax
import jax.numpy as jnp
from jax import lax
from jax.experimental import pallas as pl
from jax.experimental.pallas import tpu as pltpu
from jax.experimental.pallas import tpu_sc as plsc

N = 10000
E = 320000
DX = 128
DE = 16
G = 64
L = 3
NC = 40
EPS = 1e-5

NCORE = 2
NSUB = 16
NWORK = NCORE * NSUB
CH = 80
SUP = 16
KCH = 8
CPW = SUP * KCH
EPW = CH * CPW
E_PAD = NWORK * EPW
NHB = 2
NPB = 2
LA = 1
LAG = 1
NAGG = 10112
RPT = NAGG // NSUB

BN = 1000
BE = 2048

_f32 = jnp.float32


def _h0_body(x_ref, w_ref, scale_ref, bias_ref, o_ref):
    h = jnp.dot(x_ref[...], w_ref[...], preferred_element_type=_f32)
    o_ref[...] = jnp.maximum(h * scale_ref[...] + bias_ref[...], 0.0)


def _h0(x, w, scale, bias):
    return pl.pallas_call(
        _h0_body,
        grid=(N // BN,),
        in_specs=[
            pl.BlockSpec((BN, DX), lambda i: (i, 0)),
            pl.BlockSpec((DX, DX), lambda i: (0, 0)),
            pl.BlockSpec((1, DX), lambda i: (0, 0)),
            pl.BlockSpec((1, DX), lambda i: (0, 0)),
        ],
        out_specs=pl.BlockSpec((BN, DX), lambda i: (i, 0)),
        out_shape=jax.ShapeDtypeStruct((N, DX), _f32),
    )(x, w, scale, bias)


def _edge_body(ea_ref, wp0, wp1, wp2, we0, be0, we1, be1, p0_ref, p1_ref, p2_ref):
    ea0 = ea_ref[...]
    p0_ref[...] = jnp.dot(ea0, wp0[...], preferred_element_type=_f32)
    ea1 = jnp.dot(ea0, we0[...], preferred_element_type=_f32) + be0[...]
    p1_ref[...] = jnp.dot(ea1, wp1[...], preferred_element_type=_f32)
    ea2 = jnp.dot(ea1, we1[...], preferred_element_type=_f32) + be1[...]
    p2_ref[...] = jnp.dot(ea2, wp2[...], preferred_element_type=_f32)


def _edge_chain(ea_pad, Wp, We, be):
    ew = pl.BlockSpec((DE, DX), lambda i: (0, 0))
    sw = pl.BlockSpec((DE, DE), lambda i: (0, 0))
    bw = pl.BlockSpec((1, DE), lambda i: (0, 0))
    pspec = pl.BlockSpec((BE, DX), lambda i: (i, 0))
    pshape = jax.ShapeDtypeStruct((E_PAD, DX), _f32)
    return pl.pallas_call(
        _edge_body,
        grid=(E_PAD // BE,),
        in_specs=[pl.BlockSpec((BE, DE), lambda i: (i, 0)), ew, ew, ew,
                  sw, bw, sw, bw],
        out_specs=[pspec, pspec, pspec],
        out_shape=[pshape, pshape, pshape],
    )(ea_pad, Wp[0], Wp[1], Wp[2], We[0], be[0].reshape(1, DE),
      We[1], be[1].reshape(1, DE))


def _sc_layer_body(h_hbm, p_hbm, src_hbm, dst_hbm, zeros_hbm, out_hbm,
                   src_v, dst_v, hbuf, pbuf, agg_sh,
                   gsem, psem, ssem, isem):
    cc = lax.axis_index("c")
    ss = lax.axis_index("s")
    wid = ss * NCORE + cc
    pbase = wid * EPW
    pltpu.sync_copy(zeros_hbm.at[pl.ds(ss * RPT, RPT)],
                    agg_sh.at[pl.ds(ss * RPT, RPT)])
    plsc.subcore_barrier()

    def issue_idx(s, sl):
        pltpu.async_copy(src_hbm.at[wid, pl.ds(s * KCH, KCH)],
                         src_v.at[sl], isem)
        pltpu.async_copy(dst_hbm.at[wid, pl.ds(s * KCH, KCH)],
                         dst_v.at[sl], isem)

    def wait_idx():
        pltpu.make_async_copy(src_hbm.at[wid, pl.ds(0, KCH)],
                              src_v.at[0], isem).wait()
        pltpu.make_async_copy(dst_hbm.at[wid, pl.ds(0, KCH)],
                              dst_v.at[0], isem).wait()

    def issue_g(sl, k, hslot):
        pltpu.async_copy(h_hbm.at[src_v.at[sl, k]], hbuf.at[hslot], gsem)

    def issue_p(c, pslot):
        pltpu.async_copy(p_hbm.at[pl.ds(pbase + c * CH, CH)],
                         pbuf.at[pslot], psem)

    def wait_gp(hslot, pslot):
        pltpu.make_async_copy(h_hbm.at[src_v.at[0, 0]], hbuf.at[hslot],
                              gsem).wait()
        pltpu.make_async_copy(p_hbm.at[pl.ds(0, CH)], pbuf.at[pslot],
                              psem).wait()

    def wait_scatter(pslot):
        pltpu.make_async_copy(pbuf.at[pslot], agg_sh.at[dst_v.at[0, 0]],
                              ssem).wait()

    def compute(hslot, pslot):
        @plsc.parallel_loop(0, CH * (DX // 16), unroll=8)
        def _(j):
            r = j >> 3
            off = (j & 7) * 16
            hv = hbuf[hslot, r, pl.ds(off, 16)]
            pv = pbuf[pslot, r, pl.ds(off, 16)]
            pbuf[pslot, r, pl.ds(off, 16)] = jnp.maximum(hv + pv, 0.0)

    def do_chunk(c, sl, k, first_super, last_super):
        hslot, pslot = k % NHB, k % NPB
        if not (first_super and k < LA):
            wait_scatter((k + LA) % NPB)
        if k == KCH - LAG and not last_super:
            wait_idx()
        if not last_super or k < KCH - LAG:
            gsl = sl if k < KCH - LAG else 1 - sl
            issue_g(gsl, (k + LAG) % KCH, (k + LAG) % NHB)
        if not last_super or k < KCH - LA:
            issue_p(c + LA, (k + LA) % NPB)
        wait_gp(hslot, pslot)
        compute(hslot, pslot)
        pltpu.async_copy(pbuf.at[pslot], agg_sh.at[dst_v.at[sl, k]], ssem,
                         add=True)

    def run_super(s, sl, first_super, last_super):
        for k in range(KCH):
            do_chunk(s * KCH + k if not first_super else k,
                     sl, k, first_super, last_super)
            if k == 1 and not first_super and not last_super:
                issue_idx(s + 1, 1 - sl)

    pltpu.sync_copy(src_hbm.at[wid, pl.ds(0, KCH)], src_v.at[0])
    pltpu.sync_copy(dst_hbm.at[wid, pl.ds(0, KCH)], dst_v.at[0])
    issue_idx(1, 1)
    for c in range(LAG):
        issue_g(0, c, c % NHB)
    for c in range(LA):
        issue_p(c, c % NPB)
    run_super(0, 0, True, False)

    def super_body(s, carry):
        sl = lax.rem(s, 2)
        run_super(s, sl, False, False)
        return carry

    lax.fori_loop(1, SUP - 1, super_body, 0)
    run_super(SUP - 1, (SUP - 1) % 2, False, True)
    for k in range(LA):
        wait_scatter((KCH - LA + k) % NPB)
    plsc.subcore_barrier()
    pltpu.sync_copy(agg_sh.at[pl.ds(ss * RPT, RPT)],
                    out_hbm.at[cc, pl.ds(ss * RPT, RPT)])


def _sc_layer(h, p, src3d, dst3d, zeros):
    mesh = plsc.VectorSubcoreMesh(core_axis_name="c", subcore_axis_name="s",
                                  num_cores=NCORE, num_subcores=NSUB)
    f = pl.kernel(
        _sc_layer_body,
        out_type=jax.ShapeDtypeStruct((2, NAGG, DX), _f32),
        mesh=mesh,
        scratch_types=[
            pltpu.VMEM((2, KCH, CH), jnp.int32),
            pltpu.VMEM((2, KCH, CH), jnp.int32),
            pltpu.VMEM((NHB, CH, DX), _f32),
            pltpu.VMEM((NPB, CH, DX), _f32),
            pltpu.VMEM_SHARED((NAGG, DX), _f32),
            pltpu.SemaphoreType.DMA,
            pltpu.SemaphoreType.DMA,
            pltpu.SemaphoreType.DMA,
            pltpu.SemaphoreType.DMA,
        ],
    )
    return f(h, p, src3d, dst3d, zeros)


def _upd_body(h_ref, a0_ref, a1_ref, wc, bc, b3d, hn_ref, em_ref):
    i = pl.program_id(0)
    hn = jnp.dot(h_ref[...] + a0_ref[0] + a1_ref[0], wc[...],
                 preferred_element_type=_f32) + bc[...]
    hn_ref[...] = hn
    onehot = (b3d[0] == lax.broadcasted_iota(jnp.int32, (G, BN), 0)).astype(_f32)
    part = jnp.dot(onehot, hn, preferred_element_type=_f32)

    @pl.when(i == 0)
    def _():
        em_ref[...] = part

    @pl.when(i > 0)
    def _():
        em_ref[...] = em_ref[...] + part


def _update_pool(h, agg2, wc, bc, batch3d):
    return pl.pallas_call(
        _upd_body,
        grid=(N // BN,),
        in_specs=[
            pl.BlockSpec((BN, DX), lambda i: (i, 0)),
            pl.BlockSpec((1, BN, DX), lambda i: (0, i, 0)),
            pl.BlockSpec((1, BN, DX), lambda i: (1, i, 0)),
            pl.BlockSpec((DX, DX), lambda i: (0, 0)),
            pl.BlockSpec((1, DX), lambda i: (0, 0)),
            pl.BlockSpec((1, 1, BN), lambda i: (i, 0, 0)),
        ],
        out_specs=[pl.BlockSpec((BN, DX), lambda i: (i, 0)),
                   pl.BlockSpec((G, DX), lambda i: (0, 0))],
        out_shape=[jax.ShapeDtypeStruct((N, DX), _f32),
                   jax.ShapeDtypeStruct((G, DX), _f32)],
    )(h, agg2, agg2, wc, bc, batch3d)


def _fin_body(e0, e1, e2, s0, t0, s1, t1, s2, t2, w0, w1, w2, bl, o_ref):
    acc = bl[...]
    for eref, s, t, w in ((e0, s0, t0, w0), (e1, s1, t1, w1), (e2, s2, t2, w2)):
        v = jnp.maximum(eref[...] * s[...] + t[...], 0.0)
        acc = acc + jnp.dot(v, w[...], preferred_element_type=_f32)
    o_ref[...] = acc


def _final(embds, scales, betas, wls, bl_pad):
    return pl.pallas_call(
        _fin_body,
        out_shape=jax.ShapeDtypeStruct((G, DX), _f32),
    )(embds[0], embds[1], embds[2],
      scales[0], betas[0], scales[1], betas[1], scales[2], betas[2],
      wls[0], wls[1], wls[2], bl_pad)


def kernel(x, edge_index, edge_attr, batch, W_start, b_start, g_start,
           beta_start, Wp, Wc, bc, We, be, g_embd, beta_embd, W_lin, b_lin):
    inv = 1.0 / jnp.sqrt(jnp.float32(1.0 + EPS))
    scale0 = (g_start * inv).reshape(1, DX)
    bias0 = (b_start * g_start * inv + beta_start).reshape(1, DX)
    h = _h0(x, W_start, scale0, bias0)

    pad = E_PAD - E
    src_p = jnp.concatenate([edge_index[0], jnp.zeros((pad,), jnp.int32)])
    dst_p = jnp.concatenate([edge_index[1], jnp.full((pad,), N, jnp.int32)])
    src3d = src_p.reshape(NWORK, CPW, CH)
    dst3d = dst_p.reshape(NWORK, CPW, CH)
    ea_pad = jnp.concatenate([edge_attr, jnp.zeros((pad, DE), _f32)])
    zeros = jnp.zeros((NAGG, DX), _f32)
    batch3d = batch.reshape(N // BN, 1, BN)

    p0, p1, p2 = _edge_chain(ea_pad, Wp, We, be)
    ps = (p0, p1, p2)

    emb_scales = [(g_embd[i] * inv).reshape(1, DX) for i in range(L)]
    emb_betas = [beta_embd[i].reshape(1, DX) for i in range(L)]

    embds = []
    for i in range(L):
        agg2 = _sc_layer(h, ps[i], src3d, dst3d, zeros)
        h, em = _update_pool(h, agg2, Wc[i], bc[i].reshape(1, DX), batch3d)
        embds.append(em)

    wls = [jnp.pad(W_lin[i * DX:(i + 1) * DX], ((0, 0), (0, DX - NC)))
           for i in range(L)]
    bl_pad = jnp.pad(b_lin, (0, DX - NC)).reshape(1, DX)
    out_pad = _final(embds, emb_scales, emb_betas, wls, bl_pad)
    return out_pad[:, :NC]
